# baseline (device time: 8175 ns/iter reference)
import jax
import jax.numpy as jnp
from jax import lax
from jax.experimental import pallas as pl
from jax.experimental.pallas import tpu as pltpu

N_DEV = 4
HALF = 64


def _ce(v, j, asc, flip):
    m_rows = v.shape[0]
    if j == HALF:
        lidx = lax.broadcasted_iota(jnp.int32, v.shape, 1)
        bitj = (lidx & HALF) != 0
        partner = pltpu.roll(v, HALF, 1)
    else:
        jr = j if j < HALF else j // 2
        ridx = lax.broadcasted_iota(jnp.int32, v.shape, 0)
        bitj = (ridx & jr) != 0
        partner = jnp.where(
            bitj, pltpu.roll(v, jr, 0), pltpu.roll(v, m_rows - jr, 0)
        )
    take_min = jnp.logical_xor(jnp.logical_xor(asc, bitj), flip)
    return jnp.where(take_min, jnp.minimum(v, partner), jnp.maximum(v, partner))


def kernel(x, level=4):
    m_per, n = x.shape
    assert m_per == 2 * HALF and n == HALF
    m_rows = N_DEV * HALF

    def body(x_ref, out_ref, gather_ref, send_sems, recv_sems):
        my = lax.axis_index("i")

        if level == -1:
            out_ref[:, :] = x_ref[:, :]
            return

        barrier_sem = pltpu.get_barrier_semaphore()
        if level in (10, 11):
            peers = [1] if level == 11 else [1, 3]
            for d in peers:
                pl.semaphore_signal(
                    barrier_sem, inc=1,
                    device_id=((my + d) % N_DEV,),
                    device_id_type=pl.DeviceIdType.MESH,
                )
            pl.semaphore_wait(barrier_sem, len(peers))
            out_ref[:, :] = x_ref[:, :]
            return
        for d in range(1, N_DEV):
            pl.semaphore_signal(
                barrier_sem, inc=1,
                device_id=((my + d) % N_DEV,),
                device_id_type=pl.DeviceIdType.MESH,
            )
        pl.semaphore_wait(barrier_sem, N_DEV - 1)

        xv = x_ref[:, :]
        v = jnp.concatenate([xv[:HALF, :], xv[HALF:, :]], axis=1)

        if level == 0:
            out_ref[:, :] = xv
            return

        desc = (my % 2) == 1
        ridx = lax.broadcasted_iota(jnp.int32, v.shape, 0)
        lidx = lax.broadcasted_iota(jnp.int32, v.shape, 1)
        k = 2
        while k <= m_per:
            if k < HALF:
                asc = (ridx & k) == 0
            elif k == HALF:
                asc = (lidx & HALF) == 0
            else:
                asc = True
            j = k // 2
            while j >= 1:
                v = _ce(v, j, asc, desc)
                j //= 2
            k *= 2
        gather_ref[pl.ds(my * HALF, HALF), :] = v

        if level == 1:
            out_ref[0:HALF, :] = v[:, :HALF]
            out_ref[HALF : 2 * HALF, :] = v[:, HALF:]
            return

        rdmas = []
        for d in range(1, N_DEV):
            rdma = pltpu.make_async_remote_copy(
                src_ref=gather_ref.at[pl.ds(my * HALF, HALF)],
                dst_ref=gather_ref.at[pl.ds(my * HALF, HALF)],
                send_sem=send_sems.at[d - 1],
                recv_sem=recv_sems.at[d - 1],
                device_id=((my + d) % N_DEV,),
                device_id_type=pl.DeviceIdType.MESH,
            )
            rdma.start()
            rdmas.append(rdma)
        for rdma in rdmas:
            rdma.wait_send()

        def merge_round_256(slab, flip):
            for j in (128, 64, 32, 16, 8, 4, 2, 1):
                slab = _ce(slab, j, True, flip)
            return slab

        if level == 2:
            for rdma in rdmas:
                rdma.wait_recv()
            w = gather_ref[pl.ds(my * HALF, HALF), :]
            out_ref[0:HALF, :] = w[:, :HALF]
            out_ref[HALF : 2 * HALF, :] = w[:, HALF:]
            return

        B = my // 2
        rdmas[0].wait_recv()
        rdmas[2].wait_recv()
        near = gather_ref[pl.ds(B * 2 * HALF, 2 * HALF), :]
        near = merge_round_256(near, B == 1)
        gather_ref[pl.ds(B * 2 * HALF, 2 * HALF), :] = near

        rdmas[1].wait_recv()
        far = gather_ref[pl.ds((1 - B) * 2 * HALF, 2 * HALF), :]
        far = merge_round_256(far, B == 0)
        gather_ref[pl.ds((1 - B) * 2 * HALF, 2 * HALF), :] = far

        if level == 3:
            w = gather_ref[pl.ds(my * HALF, HALF), :]
            out_ref[0:HALF, :] = w[:, :HALF]
            out_ref[HALF : 2 * HALF, :] = w[:, HALF:]
            return

        v = gather_ref[:, :]
        v = _ce(v, 256, True, False)
        v = _ce(v, 128, True, False)
        gather_ref[:, :] = v

        w = gather_ref[pl.ds(my * HALF, HALF), :]
        for j in (64, 32, 16, 8, 4, 2, 1):
            w = _ce(w, j, True, False)

        out_ref[0:HALF, :] = w[:, :HALF]
        out_ref[HALF : 2 * HALF, :] = w[:, HALF:]

    return pl.pallas_call(
        body,
        out_shape=jax.ShapeDtypeStruct((m_per, n), x.dtype),
        in_specs=[pl.BlockSpec(memory_space=pltpu.VMEM)],
        out_specs=pl.BlockSpec(memory_space=pltpu.VMEM),
        scratch_shapes=[
            pltpu.VMEM((m_rows, 2 * HALF), x.dtype),
            pltpu.SemaphoreType.DMA((N_DEV - 1,)),
            pltpu.SemaphoreType.DMA((N_DEV - 1,)),
        ],
        compiler_params=(
            None if level == -1 else pltpu.CompilerParams(collective_id=0)
        ),
    )(x)
